# baseline (speedup 1.0000x reference)
import functools

import jax
import jax.numpy as jnp
from jax.experimental import pallas as pl


_BM_CANDIDATES = (512, 448, 392, 256, 196, 128, 112, 64, 56, 16, 8)


def _pick_bm(m):
    for bm in _BM_CANDIDATES:
        if m % bm == 0:
            return bm
    return m


def _mm_body(a_ref, b_ref, bias_ref, o_ref):
    o_ref[0] = (
        jnp.dot(a_ref[0], b_ref[0], preferred_element_type=jnp.float32)
        + bias_ref[0]
    )


def _mm(a, b, bias=None):
    g, m, k = a.shape
    n = b.shape[2]
    if bias is None:
        bias = jnp.zeros((g, 1, n), jnp.float32)
    bm = _pick_bm(m)
    return pl.pallas_call(
        _mm_body,
        grid=(g, m // bm),
        in_specs=[
            pl.BlockSpec((1, bm, k), lambda gi, mi: (gi, mi, 0)),
            pl.BlockSpec((1, k, n), lambda gi, mi: (gi, 0, 0)),
            pl.BlockSpec((1, 1, n), lambda gi, mi: (gi, 0, 0)),
        ],
        out_specs=pl.BlockSpec((1, bm, n), lambda gi, mi: (gi, mi, 0)),
        out_shape=jax.ShapeDtypeStruct((g, m, n), jnp.float32),
    )(a, b, bias)


def _gmm_body(a_ref, b_ref, o_ref):
    o_ref[0] = jax.lax.dot_general(
        a_ref[0], b_ref[0], (((0,), (0,)), ((), ())),
        preferred_element_type=jnp.float32,
    )


def _gmm(a, b):
    g, k, m = a.shape
    n = b.shape[2]
    return pl.pallas_call(
        _gmm_body,
        grid=(g,),
        in_specs=[
            pl.BlockSpec((1, k, m), lambda gi: (gi, 0, 0)),
            pl.BlockSpec((1, k, n), lambda gi: (gi, 0, 0)),
        ],
        out_specs=pl.BlockSpec((1, m, n), lambda gi: (gi, 0, 0)),
        out_shape=jax.ShapeDtypeStruct((g, m, n), jnp.float32),
    )(a, b)


def _attn_body(scale, q_ref, k_ref, v_ref, o_ref, cs_ref):
    rb = pl.program_id(1)
    s = jax.lax.dot_general(
        q_ref[0], k_ref[0], (((1,), (1,)), ((), ())),
        preferred_element_type=jnp.float32,
    ) * scale
    mx = jnp.max(s, axis=1, keepdims=True)
    p = jnp.exp(s - mx)
    l = jnp.sum(p, axis=1, keepdims=True)
    pn = p / l
    o_ref[0] = jnp.dot(pn, v_ref[0], preferred_element_type=jnp.float32)
    col = jnp.sum(pn, axis=0, keepdims=True)

    @pl.when(rb == 0)
    def _init():
        cs_ref[0] = col

    @pl.when(rb > 0)
    def _acc():
        cs_ref[0] = cs_ref[0] + col


def _attn(q, k, v, scale):
    h, n, d = q.shape
    bm = _pick_bm(n)
    return pl.pallas_call(
        functools.partial(_attn_body, scale),
        grid=(h, n // bm),
        in_specs=[
            pl.BlockSpec((1, bm, d), lambda hi, mi: (hi, mi, 0)),
            pl.BlockSpec((1, n, d), lambda hi, mi: (hi, 0, 0)),
            pl.BlockSpec((1, n, d), lambda hi, mi: (hi, 0, 0)),
        ],
        out_specs=[
            pl.BlockSpec((1, bm, d), lambda hi, mi: (hi, mi, 0)),
            pl.BlockSpec((1, 1, n), lambda hi, mi: (hi, 0, 0)),
        ],
        out_shape=[
            jax.ShapeDtypeStruct((h, n, d), jnp.float32),
            jax.ShapeDtypeStruct((h, 1, n), jnp.float32),
        ],
    )(q, k, v)


def _topk_body(kf, chunk, cs_row_ref, cs_col_ref, oh_ref):
    n = cs_row_ref.shape[2]
    vr = cs_row_ref[0]
    iota_i = jax.lax.broadcasted_iota(jnp.int32, (chunk, n), 1)
    iota_j = jax.lax.broadcasted_iota(jnp.int32, (chunk, n), 0)
    iota_r = jax.lax.broadcasted_iota(jnp.int32, (chunk, kf), 1)

    def body(c, _):
        vj = cs_col_ref[0, pl.ds(c * chunk, chunk), :]
        jglob = c * chunk + iota_j
        beat = (vr > vj) | ((vr == vj) & (iota_i < jglob))
        rank = jnp.sum(beat.astype(jnp.int32), axis=1, keepdims=True)
        oh_blk = jnp.where((rank == iota_r) & (rank < kf), 1.0, 0.0)
        oh_ref[0, pl.ds(c * chunk, chunk), :] = oh_blk.astype(jnp.float32)
        return 0

    jax.lax.fori_loop(0, n // chunk, body, 0)


def _topk_onehot(cs, kf):
    h, _, n = cs.shape
    cs_col = jnp.transpose(cs, (0, 2, 1))
    chunk = _pick_bm(n)
    return pl.pallas_call(
        functools.partial(_topk_body, kf, chunk),
        grid=(h,),
        in_specs=[
            pl.BlockSpec((1, 1, n), lambda hi: (hi, 0, 0)),
            pl.BlockSpec((1, n, 1), lambda hi: (hi, 0, 0)),
        ],
        out_specs=pl.BlockSpec((1, n, kf), lambda hi: (hi, 0, 0)),
        out_shape=jax.ShapeDtypeStruct((h, n, kf), jnp.float32),
    )(cs, cs_col)


def kernel(x, W_embed, b_embed, W_down, b_down, W_up, b_up, W_qkv_c, b_qkv_c, W_qkv_t, b_qkv_t):
    hd = 64
    scale = hd ** (-0.5)
    dim = W_embed.shape[0]
    nh = dim // hd
    H2 = x.shape[2] // 2
    h = H2 // 2
    n = h * h
    n2 = H2 * H2
    kf = max(1, n // 4)

    xp = (
        x[0]
        .reshape(3, H2, 2, H2, 2)
        .transpose(1, 3, 0, 2, 4)
        .reshape(n2, 12)
    )
    we = W_embed.reshape(dim, 12).T
    xe_tok = _mm(xp[None], we[None], b_embed.reshape(1, 1, dim))[0]
    xe_img = xe_tok.reshape(H2, H2, dim)

    xe_pad = jnp.pad(xe_img, ((1, 1), (1, 1), (0, 0)))
    slices = [
        xe_pad[ki:ki + 2 * h:2, kj:kj + 2 * h:2]
        for ki in range(4)
        for kj in range(4)
    ]
    a_down = jnp.stack(slices, axis=2).reshape(n, 16 * dim)
    w_down = W_down.transpose(2, 3, 1, 0).reshape(16 * dim, dim)
    xd_tok = _mm(a_down[None], w_down[None], b_down.reshape(1, 1, dim))[0]

    tokens = xd_tok.reshape(n, nh, hd).transpose(1, 0, 2)
    wqc = jnp.broadcast_to(W_qkv_c.T[None], (nh, hd, 3 * hd))
    bqc = jnp.broadcast_to(b_qkv_c.reshape(1, 1, 3 * hd), (nh, 1, 3 * hd))
    qkv = _mm(tokens, wqc, bqc)
    q, k, v = qkv[..., :hd], qkv[..., hd:2 * hd], qkv[..., 2 * hd:]
    out1, cs = _attn(q, k, v, scale)

    out_img = out1.transpose(1, 0, 2).reshape(h, h, dim)
    op = jnp.pad(out_img, ((1, 1), (1, 1), (0, 0)))
    taps = {0: ((0, -1), (2, 0)), 1: ((1, 0), (3, 1))}
    a_cls = []
    w_cls = []
    for pa in (0, 1):
        for pb in (0, 1):
            a_cls.append(
                jnp.concatenate(
                    [
                        op[1 + da:1 + da + h, 1 + db:1 + db + h]
                        for (ki, da) in taps[pa]
                        for (kj, db) in taps[pb]
                    ],
                    axis=-1,
                ).reshape(n, 4 * dim)
            )
            w_cls.append(
                jnp.concatenate(
                    [
                        W_up[:, :, 3 - ki, 3 - kj]
                        for (ki, da) in taps[pa]
                        for (kj, db) in taps[pb]
                    ],
                    axis=0,
                )
            )
    y_cls = _mm(
        jnp.stack(a_cls),
        jnp.stack(w_cls),
        jnp.broadcast_to(b_up.reshape(1, 1, dim), (4, 1, dim)),
    )
    coarse_img = (
        y_cls.reshape(2, 2, h, h, dim)
        .transpose(2, 0, 3, 1, 4)
        .reshape(H2, H2, dim)
    )

    patches = (
        xe_img.reshape(h, 2, h, 2, nh, hd)
        .transpose(4, 0, 2, 1, 3, 5)
        .reshape(nh, n, 4 * hd)
    )
    oht = _topk_onehot(cs, kf)
    sel = _gmm(oht, patches)
    tok2 = sel.reshape(nh, kf * 4, hd)
    wqt = jnp.broadcast_to(W_qkv_t.T[None], (nh, hd, 3 * hd))
    bqt = jnp.broadcast_to(b_qkv_t.reshape(1, 1, 3 * hd), (nh, 1, 3 * hd))
    qkv2 = _mm(tok2, wqt, bqt)
    q2, k2, v2 = qkv2[..., :hd], qkv2[..., hd:2 * hd], qkv2[..., 2 * hd:]
    out2, _ = _attn(q2, k2, v2, scale)
    delta = (out2 - tok2).reshape(nh, kf, 4 * hd)
    scat = _mm(oht, delta)
    scat_img = (
        scat.reshape(nh, h, h, 2, 2, hd)
        .transpose(1, 3, 2, 4, 0, 5)
        .reshape(H2, H2, dim)
    )

    final = 2.0 * xe_img + coarse_img + scat_img
    return jnp.transpose(final, (2, 0, 1))[None]
